# baseline (speedup 1.0000x reference)
import jax
import jax.numpy as jnp
from jax import lax
from jax.experimental import pallas as pl
from jax.experimental.pallas import tpu as pltpu
from jax.experimental.pallas import tpu_sc as plsc

NUM_CATEGORIES = 1000000
L = 26
D = 16
B = 16384

NC = 2
NS = 16
NW = NC * NS
PAIRS = L * D
PPW = PAIRS // NW

IDX_ROW = 128
ROWS_B = B // IDX_ROW
VECS_B = B // 16


def _body(xt_hbm, table_hbm, pw_hbm, pb_hbm, out_hbm,
          idx_v, buf_v, pw_v, pb_v, gsem):
    wid = lax.axis_index("s") * NC + lax.axis_index("c")
    p0 = wid * PPW
    l0 = p0 // D

    pltpu.sync_copy(xt_hbm.at[l0], idx_v.at[0])
    l_last = (p0 + PPW - 1) // D

    @pl.when(l_last != l0)
    def _():
        pltpu.sync_copy(xt_hbm.at[l_last], idx_v.at[1])

    pltpu.sync_copy(pw_hbm, pw_v)
    pltpu.sync_copy(pb_hbm, pb_v)

    def fire(k, slot):
        pair = p0 + k
        l = pair // D
        d = pair % D
        rel = l - l0

        def fire_one(r, carry):
            pltpu.async_copy(
                table_hbm.at[d].at[idx_v.at[rel, r]],
                buf_v.at[slot, pl.ds(r * IDX_ROW, IDX_ROW)],
                gsem.at[slot])
            return carry
        lax.fori_loop(0, ROWS_B, fire_one, 0)

    def drain(k, slot):
        pair = p0 + k
        pltpu.make_async_copy(
            out_hbm.at[pair // D, pair % D], buf_v.at[slot],
            gsem.at[slot]).wait()

    fire(0, 0)
    for k in range(PPW):
        slot = k % 2
        if k + 1 < PPW:
            fire(k + 1, 1 - slot)
        drain(k, slot)

        pair = p0 + k
        l = pair // D
        d = pair % D
        lvec = jnp.full((16,), l, jnp.int32)
        dvec = jnp.full((16,), d, jnp.int32)
        w = plsc.load_gather(pw_v, [lvec, dvec])
        b = plsc.load_gather(pb_v, [lvec, dvec])

        def affine(i, carry):
            for u in range(4):
                sl = pl.ds((i * 4 + u) * 16, 16)
                buf_v[slot, sl] = buf_v[slot, sl] * w + b
            return carry
        lax.fori_loop(0, VECS_B // 4, affine, 0)

        pltpu.sync_copy(buf_v.at[slot], out_hbm.at[l, d])


@jax.jit
def kernel(x, shared_embed, position_weights, position_bias):
    xt = x.T.reshape(L, ROWS_B, IDX_ROW)
    table_t = shared_embed.T
    mesh = plsc.VectorSubcoreMesh(core_axis_name="c", subcore_axis_name="s")
    out_p = pl.kernel(
        _body,
        out_type=jax.ShapeDtypeStruct((L, D, B), jnp.float32),
        mesh=mesh,
        compiler_params=pltpu.CompilerParams(
            use_tc_tiling_on_sc=False, needs_layout_passes=False),
        scratch_types=[
            pltpu.VMEM((2, ROWS_B, IDX_ROW), jnp.int32),
            pltpu.VMEM((2, B), jnp.float32),
            pltpu.VMEM((L, D), jnp.float32),
            pltpu.VMEM((L, D), jnp.float32),
            pltpu.SemaphoreType.DMA((2,)),
        ],
    )(xt, table_t, position_weights, position_bias)
    return out_p.transpose(2, 0, 1)
